# baseline (speedup 1.0000x reference)
import functools

import jax
import jax.numpy as jnp
from jax import lax
from jax.experimental import pallas as pl
from jax.experimental.pallas import tpu as pltpu
from jax.experimental.pallas import tpu_sc as plsc

N = 10000
E = 320000
D_IN = 128
D_HID = 128
D_OUT = 47
D_OUT_PAD = 128

NC = 2
NS = 16
NW = NC * NS
CHUNK = 128

GCH = 128
NCH = 80
NCH_ALLOC = 81
SPAN = NCH_ALLOC * GCH
E_PER_W = 79 * GCH
E_ALLOC = NW * SPAN

ACC_ROWS = 10240
ZROWS_PER_TILE = ACC_ROWS // NS


def _make_agg(width):
    mesh = plsc.VectorSubcoreMesh(core_axis_name="c", subcore_axis_name="s")

    @functools.partial(
        pl.kernel,
        out_type=jax.ShapeDtypeStruct((NC * ACC_ROWS, width), jnp.float32),
        mesh=mesh,
        scratch_types=[
            pltpu.VMEM((2, GCH), jnp.int32),
            pltpu.VMEM((2, GCH), jnp.int32),
            pltpu.VMEM((2, GCH, width), jnp.float32),
            pltpu.VMEM_SHARED((ACC_ROWS, width), jnp.float32),
            [pltpu.SemaphoreType.DMA] * 2,
        ],
    )
    def agg(p_hbm, src_hbm, dst_hbm, zeros_hbm, out_hbm,
            s_ring, d_ring, bufs, acc, sem_g):
        c = lax.axis_index("c")
        s = lax.axis_index("s")
        ebase = (c * NS + s) * SPAN

        def src_at(g):
            return src_hbm.at[pl.ds(ebase + g * GCH, GCH)]

        def dst_at(g):
            return dst_hbm.at[pl.ds(ebase + g * GCH, GCH)]

        pltpu.sync_copy(zeros_hbm, bufs.at[0])
        z0 = s * ZROWS_PER_TILE
        for j in range(ZROWS_PER_TILE // GCH):
            pltpu.sync_copy(bufs.at[0], acc.at[pl.ds(z0 + j * GCH, GCH)])
        plsc.subcore_barrier()

        pltpu.sync_copy(src_at(0), s_ring.at[0])
        pltpu.sync_copy(dst_at(0), d_ring.at[0])
        pltpu.async_copy(p_hbm.at[s_ring.at[0]], bufs.at[0], sem_g[0])

        def body(i2, carry):
            for off in range(2):
                g = i2 * 2 + off
                cur = off
                nxt = 1 - off
                pltpu.sync_copy(src_at(g + 1), s_ring.at[nxt])
                pltpu.sync_copy(dst_at(g + 1), d_ring.at[nxt])
                pltpu.make_async_copy(p_hbm.at[s_ring.at[cur]],
                                      bufs.at[cur], sem_g[cur]).wait()
                pltpu.async_copy(p_hbm.at[s_ring.at[nxt]], bufs.at[nxt],
                                 sem_g[nxt])
                pltpu.sync_copy(bufs.at[cur], acc.at[d_ring.at[cur]],
                                add=True)
            return carry

        lax.fori_loop(0, NCH // 2, body, 0)
        pltpu.make_async_copy(p_hbm.at[s_ring.at[0]], bufs.at[0],
                              sem_g[0]).wait()
        plsc.subcore_barrier()

        for j in range(ZROWS_PER_TILE // GCH):
            pltpu.sync_copy(acc.at[pl.ds(z0 + j * GCH, GCH)], bufs.at[0])
            pltpu.sync_copy(bufs.at[0],
                            out_hbm.at[pl.ds(c * ACC_ROWS + z0 + j * GCH,
                                             GCH)])

    return agg


CH_DEG = 80


def _make_deg():
    mesh = plsc.VectorSubcoreMesh(core_axis_name="c", subcore_axis_name="s")

    @functools.partial(
        pl.kernel,
        out_type=jax.ShapeDtypeStruct((NC * ACC_ROWS, D_HID), jnp.float32),
        mesh=mesh,
        scratch_types=[
            pltpu.VMEM((CH_DEG, CHUNK), jnp.int32),
            pltpu.VMEM((CHUNK, D_HID), jnp.float32),
            pltpu.VMEM_SHARED((ACC_ROWS, D_HID), jnp.float32),
            pltpu.SemaphoreType.DMA,
        ],
    )
    def deg(ones_hbm, zeros_hbm, dst_hbm, out_hbm, dst_i, ones_v, acc, sem):
        c = lax.axis_index("c")
        s = lax.axis_index("s")
        wid = c * NS + s

        pltpu.sync_copy(dst_hbm.at[pl.ds(wid * CH_DEG, CH_DEG)], dst_i)
        pltpu.sync_copy(zeros_hbm, ones_v)
        z0 = s * ZROWS_PER_TILE
        drem = ZROWS_PER_TILE % CHUNK
        dzr0 = z0 + (ZROWS_PER_TILE // CHUNK) * CHUNK
        for j in range(ZROWS_PER_TILE // CHUNK):
            pltpu.sync_copy(ones_v, acc.at[pl.ds(z0 + j * CHUNK, CHUNK)])
        if drem:
            pltpu.sync_copy(ones_v.at[pl.ds(0, drem)],
                            acc.at[pl.ds(dzr0, drem)])
        pltpu.sync_copy(ones_hbm, ones_v)
        plsc.subcore_barrier()

        DB = 8

        def body(blk, carry):
            g0 = blk * DB
            for b in range(DB):
                pltpu.async_copy(ones_v, acc.at[dst_i.at[g0 + b]], sem,
                                 add=True)
            for b in range(DB):
                pltpu.make_async_copy(ones_v, acc.at[dst_i.at[g0]], sem).wait()
            return carry

        lax.fori_loop(0, CH_DEG // DB, body, 0)
        plsc.subcore_barrier()

        for j in range(ZROWS_PER_TILE // CHUNK):
            pltpu.sync_copy(acc.at[pl.ds(z0 + j * CHUNK, CHUNK)], ones_v)
            pltpu.sync_copy(ones_v,
                            out_hbm.at[pl.ds(c * ACC_ROWS + z0 + j * CHUNK,
                                             CHUNK)])
        if drem:
            pltpu.sync_copy(acc.at[pl.ds(dzr0, drem)],
                            ones_v.at[pl.ds(0, drem)])
            pltpu.sync_copy(ones_v.at[pl.ds(0, drem)],
                            out_hbm.at[pl.ds(c * ACC_ROWS + dzr0, drem)])

    return deg


_BN = 2000


def _tc_first_body(x_ref, wn_ref, ws_ref, b_ref, p_ref, s_ref):
    xb = x_ref[...]
    p_ref[...] = jnp.dot(xb, wn_ref[...], preferred_element_type=jnp.float32)
    s_ref[...] = (jnp.dot(xb, ws_ref[...], preferred_element_type=jnp.float32)
                  + b_ref[...])


def _tc_first(x, wn, ws, b):
    return pl.pallas_call(
        _tc_first_body,
        grid=(N // _BN,),
        in_specs=[
            pl.BlockSpec((_BN, D_IN), lambda i: (i, 0)),
            pl.BlockSpec((D_IN, D_HID), lambda i: (0, 0)),
            pl.BlockSpec((D_IN, D_HID), lambda i: (0, 0)),
            pl.BlockSpec((1, D_HID), lambda i: (0, 0)),
        ],
        out_specs=[
            pl.BlockSpec((_BN, D_HID), lambda i: (i, 0)),
            pl.BlockSpec((_BN, D_HID), lambda i: (i, 0)),
        ],
        out_shape=[
            jax.ShapeDtypeStruct((N, D_HID), jnp.float32),
            jax.ShapeDtypeStruct((N, D_HID), jnp.float32),
        ],
    )(x, wn, ws, b)


def _tc_mid_body(s_ref, agg_ref, deg_ref, wn_ref, ws_ref, b_ref,
                 p_ref, so_ref):
    a = agg_ref[0] + agg_ref[1]
    dcnt = deg_ref[0, :, 0:1] + deg_ref[1, :, 0:1]
    d = jnp.maximum(dcnt, 1.0)
    h = jnp.maximum(s_ref[...] + a / d, 0.0)
    p_ref[...] = jnp.dot(h, wn_ref[...], preferred_element_type=jnp.float32)
    so_ref[...] = (jnp.dot(h, ws_ref[...], preferred_element_type=jnp.float32)
                   + b_ref[...])


def _tc_mid(s_prev, agg, deg, wn, ws, b, d_out):
    return pl.pallas_call(
        _tc_mid_body,
        grid=(N // _BN,),
        in_specs=[
            pl.BlockSpec((_BN, D_HID), lambda i: (i, 0)),
            pl.BlockSpec((NC, _BN, D_HID), lambda i: (0, i, 0)),
            pl.BlockSpec((NC, _BN, D_HID), lambda i: (0, i, 0)),
            pl.BlockSpec((D_HID, d_out), lambda i: (0, 0)),
            pl.BlockSpec((D_HID, d_out), lambda i: (0, 0)),
            pl.BlockSpec((1, d_out), lambda i: (0, 0)),
        ],
        out_specs=[
            pl.BlockSpec((_BN, d_out), lambda i: (i, 0)),
            pl.BlockSpec((_BN, d_out), lambda i: (i, 0)),
        ],
        out_shape=[
            jax.ShapeDtypeStruct((N, d_out), jnp.float32),
            jax.ShapeDtypeStruct((N, d_out), jnp.float32),
        ],
    )(s_prev, agg, deg, wn, ws, b)


def _tc_last_body(s_ref, agg_ref, deg_ref, o_ref):
    a = agg_ref[0] + agg_ref[1]
    dcnt = deg_ref[0, :, 0:1] + deg_ref[1, :, 0:1]
    d = jnp.maximum(dcnt, 1.0)
    o_ref[...] = s_ref[...] + a / d


def _tc_last(s_prev, agg, deg):
    return pl.pallas_call(
        _tc_last_body,
        grid=(N // _BN,),
        in_specs=[
            pl.BlockSpec((_BN, D_OUT_PAD), lambda i: (i, 0)),
            pl.BlockSpec((NC, _BN, D_OUT_PAD), lambda i: (0, i, 0)),
            pl.BlockSpec((NC, _BN, D_HID), lambda i: (0, i, 0)),
        ],
        out_specs=pl.BlockSpec((_BN, D_OUT_PAD), lambda i: (i, 0)),
        out_shape=jax.ShapeDtypeStruct((N, D_OUT_PAD), jnp.float32),
    )(s_prev, agg, deg)


def kernel(x, edge_index, W_self0, W_neigh0, b0,
           W_self1, W_neigh1, b1, W_self2, W_neigh2, b2):
    src = edge_index[0].astype(jnp.int32)
    dst = edge_index[1].astype(jnp.int32)
    pad = 0

    def to_flat(a, fill):
        flat = jnp.concatenate(
            [a, jnp.full((NW * E_PER_W - E,), fill, jnp.int32)])
        blocks = flat.reshape(NW, E_PER_W)
        trash = jnp.full((NW, SPAN - E_PER_W), fill, jnp.int32)
        return jnp.concatenate([blocks, trash], axis=1).reshape(NW * SPAN)

    src_p = to_flat(src, 0)
    dst_p = to_flat(dst, N)
    pad_deg = NW * CH_DEG * CHUNK - E
    dst_deg = jnp.concatenate(
        [dst, jnp.full((pad_deg,), N, jnp.int32)]).reshape(
            NW * CH_DEG, CHUNK)

    zeros64 = jnp.zeros((GCH, D_HID), jnp.float32)
    zeros128 = jnp.zeros((CHUNK, D_HID), jnp.float32)
    ones128 = jnp.ones((CHUNK, D_HID), jnp.float32)

    wn2 = jnp.pad(W_neigh2, ((0, 0), (0, D_OUT_PAD - D_OUT)))
    ws2 = jnp.pad(W_self2, ((0, 0), (0, D_OUT_PAD - D_OUT)))
    b2p = jnp.pad(b2, (0, D_OUT_PAD - D_OUT)).reshape(1, D_OUT_PAD)

    deg_parts = _make_deg()(ones128, zeros128, dst_deg)
    deg3 = deg_parts.reshape(NC, ACC_ROWS, D_HID)

    agg128 = _make_agg(D_HID)

    p0, s0 = _tc_first(x, W_neigh0, W_self0, b0.reshape(1, D_HID))
    a0 = agg128(p0, src_p, dst_p, zeros64).reshape(NC, ACC_ROWS, D_HID)

    p1, s1 = _tc_mid(s0, a0, deg3, W_neigh1, W_self1,
                     b1.reshape(1, D_HID), D_HID)
    a1 = agg128(p1, src_p, dst_p, zeros64).reshape(NC, ACC_ROWS, D_HID)

    p2, s2 = _tc_mid(s1, a1, deg3, wn2, ws2, b2p, D_OUT_PAD)
    a2 = agg128(p2, src_p, dst_p, zeros64).reshape(NC, ACC_ROWS, D_OUT_PAD)

    out = _tc_last(s2, a2, deg3)
    return out[:, :D_OUT]


# speedup vs baseline: 1.1131x; 1.1131x over previous
import functools

import jax
import jax.numpy as jnp
from jax import lax
from jax.experimental import pallas as pl
from jax.experimental.pallas import tpu as pltpu
from jax.experimental.pallas import tpu_sc as plsc

N = 10000
E = 320000
D_IN = 128
D_HID = 128
D_OUT = 47
D_OUT_PAD = 128

NC = 2
NS = 16
NW = NC * NS
CHUNK = 128

GCH = 128
NCH = 80
NCH_ALLOC = 81
SPAN = NCH_ALLOC * GCH
E_PER_W = 79 * GCH
E_ALLOC = NW * SPAN

ACC_ROWS = 10240
ZROWS_PER_TILE = ACC_ROWS // NS


def _make_agg(width):
    mesh = plsc.VectorSubcoreMesh(core_axis_name="c", subcore_axis_name="s")

    @functools.partial(
        pl.kernel,
        out_type=jax.ShapeDtypeStruct((NC * ACC_ROWS, width), jnp.float32),
        mesh=mesh,
        scratch_types=[
            pltpu.VMEM((GCH,), jnp.int32),
            pltpu.VMEM((GCH,), jnp.int32),
            pltpu.VMEM((GCH, width), jnp.float32),
            pltpu.VMEM_SHARED((ACC_ROWS, width), jnp.float32),
            pltpu.SemaphoreType.DMA,
        ],
    )
    def agg(p_hbm, src_hbm, dst_hbm, zeros_hbm, out_hbm,
            src_v, dst_v, rows_v, acc, sem):
        c = lax.axis_index("c")
        s = lax.axis_index("s")
        ebase = (c * NS + s) * SPAN

        pltpu.sync_copy(zeros_hbm, rows_v)
        z0 = s * ZROWS_PER_TILE
        for j in range(ZROWS_PER_TILE // GCH):
            pltpu.sync_copy(rows_v, acc.at[pl.ds(z0 + j * GCH, GCH)])
        plsc.subcore_barrier()

        def body(g, carry):
            eb = ebase + g * GCH
            pltpu.sync_copy(src_hbm.at[pl.ds(eb, GCH)], src_v)
            pltpu.sync_copy(dst_hbm.at[pl.ds(eb, GCH)], dst_v)
            pltpu.async_copy(p_hbm.at[src_v], rows_v, sem).wait()
            pltpu.sync_copy(rows_v, acc.at[dst_v], add=True)
            return carry

        lax.fori_loop(0, NCH, body, 0)
        plsc.subcore_barrier()

        for j in range(ZROWS_PER_TILE // GCH):
            pltpu.sync_copy(acc.at[pl.ds(z0 + j * GCH, GCH)], rows_v)
            pltpu.sync_copy(rows_v,
                            out_hbm.at[pl.ds(c * ACC_ROWS + z0 + j * GCH,
                                             GCH)])

    return agg


CH_DEG = 80


def _make_deg():
    mesh = plsc.VectorSubcoreMesh(core_axis_name="c", subcore_axis_name="s")

    @functools.partial(
        pl.kernel,
        out_type=jax.ShapeDtypeStruct((NC * ACC_ROWS, D_HID), jnp.float32),
        mesh=mesh,
        scratch_types=[
            pltpu.VMEM((CH_DEG, CHUNK), jnp.int32),
            pltpu.VMEM((CHUNK, D_HID), jnp.float32),
            pltpu.VMEM_SHARED((ACC_ROWS, D_HID), jnp.float32),
            pltpu.SemaphoreType.DMA,
        ],
    )
    def deg(ones_hbm, zeros_hbm, dst_hbm, out_hbm, dst_i, ones_v, acc, sem):
        c = lax.axis_index("c")
        s = lax.axis_index("s")
        wid = c * NS + s

        pltpu.sync_copy(dst_hbm.at[pl.ds(wid * CH_DEG, CH_DEG)], dst_i)
        pltpu.sync_copy(zeros_hbm, ones_v)
        z0 = s * ZROWS_PER_TILE
        drem = ZROWS_PER_TILE % CHUNK
        dzr0 = z0 + (ZROWS_PER_TILE // CHUNK) * CHUNK
        for j in range(ZROWS_PER_TILE // CHUNK):
            pltpu.sync_copy(ones_v, acc.at[pl.ds(z0 + j * CHUNK, CHUNK)])
        if drem:
            pltpu.sync_copy(ones_v.at[pl.ds(0, drem)],
                            acc.at[pl.ds(dzr0, drem)])
        pltpu.sync_copy(ones_hbm, ones_v)
        plsc.subcore_barrier()

        DB = 8

        def body(blk, carry):
            g0 = blk * DB
            for b in range(DB):
                pltpu.async_copy(ones_v, acc.at[dst_i.at[g0 + b]], sem,
                                 add=True)
            for b in range(DB):
                pltpu.make_async_copy(ones_v, acc.at[dst_i.at[g0]], sem).wait()
            return carry

        lax.fori_loop(0, CH_DEG // DB, body, 0)
        plsc.subcore_barrier()

        for j in range(ZROWS_PER_TILE // CHUNK):
            pltpu.sync_copy(acc.at[pl.ds(z0 + j * CHUNK, CHUNK)], ones_v)
            pltpu.sync_copy(ones_v,
                            out_hbm.at[pl.ds(c * ACC_ROWS + z0 + j * CHUNK,
                                             CHUNK)])
        if drem:
            pltpu.sync_copy(acc.at[pl.ds(dzr0, drem)],
                            ones_v.at[pl.ds(0, drem)])
            pltpu.sync_copy(ones_v.at[pl.ds(0, drem)],
                            out_hbm.at[pl.ds(c * ACC_ROWS + dzr0, drem)])

    return deg


_BN = 2000


def _tc_first_body(x_ref, wn_ref, ws_ref, b_ref, p_ref, s_ref):
    xb = x_ref[...]
    p_ref[...] = jnp.dot(xb, wn_ref[...], preferred_element_type=jnp.float32)
    s_ref[...] = (jnp.dot(xb, ws_ref[...], preferred_element_type=jnp.float32)
                  + b_ref[...])


def _tc_first(x, wn, ws, b):
    return pl.pallas_call(
        _tc_first_body,
        grid=(N // _BN,),
        in_specs=[
            pl.BlockSpec((_BN, D_IN), lambda i: (i, 0)),
            pl.BlockSpec((D_IN, D_HID), lambda i: (0, 0)),
            pl.BlockSpec((D_IN, D_HID), lambda i: (0, 0)),
            pl.BlockSpec((1, D_HID), lambda i: (0, 0)),
        ],
        out_specs=[
            pl.BlockSpec((_BN, D_HID), lambda i: (i, 0)),
            pl.BlockSpec((_BN, D_HID), lambda i: (i, 0)),
        ],
        out_shape=[
            jax.ShapeDtypeStruct((N, D_HID), jnp.float32),
            jax.ShapeDtypeStruct((N, D_HID), jnp.float32),
        ],
    )(x, wn, ws, b)


def _tc_mid_body(s_ref, agg_ref, deg_ref, wn_ref, ws_ref, b_ref,
                 p_ref, so_ref):
    a = agg_ref[0] + agg_ref[1]
    dcnt = deg_ref[0, :, 0:1] + deg_ref[1, :, 0:1]
    d = jnp.maximum(dcnt, 1.0)
    h = jnp.maximum(s_ref[...] + a / d, 0.0)
    p_ref[...] = jnp.dot(h, wn_ref[...], preferred_element_type=jnp.float32)
    so_ref[...] = (jnp.dot(h, ws_ref[...], preferred_element_type=jnp.float32)
                   + b_ref[...])


def _tc_mid(s_prev, agg, deg, wn, ws, b, d_out):
    return pl.pallas_call(
        _tc_mid_body,
        grid=(N // _BN,),
        in_specs=[
            pl.BlockSpec((_BN, D_HID), lambda i: (i, 0)),
            pl.BlockSpec((NC, _BN, D_HID), lambda i: (0, i, 0)),
            pl.BlockSpec((NC, _BN, D_HID), lambda i: (0, i, 0)),
            pl.BlockSpec((D_HID, d_out), lambda i: (0, 0)),
            pl.BlockSpec((D_HID, d_out), lambda i: (0, 0)),
            pl.BlockSpec((1, d_out), lambda i: (0, 0)),
        ],
        out_specs=[
            pl.BlockSpec((_BN, d_out), lambda i: (i, 0)),
            pl.BlockSpec((_BN, d_out), lambda i: (i, 0)),
        ],
        out_shape=[
            jax.ShapeDtypeStruct((N, d_out), jnp.float32),
            jax.ShapeDtypeStruct((N, d_out), jnp.float32),
        ],
    )(s_prev, agg, deg, wn, ws, b)


def _tc_last_body(s_ref, agg_ref, deg_ref, o_ref):
    a = agg_ref[0] + agg_ref[1]
    dcnt = deg_ref[0, :, 0:1] + deg_ref[1, :, 0:1]
    d = jnp.maximum(dcnt, 1.0)
    o_ref[...] = s_ref[...] + a / d


def _tc_last(s_prev, agg, deg):
    return pl.pallas_call(
        _tc_last_body,
        grid=(N // _BN,),
        in_specs=[
            pl.BlockSpec((_BN, D_OUT_PAD), lambda i: (i, 0)),
            pl.BlockSpec((NC, _BN, D_OUT_PAD), lambda i: (0, i, 0)),
            pl.BlockSpec((NC, _BN, D_HID), lambda i: (0, i, 0)),
        ],
        out_specs=pl.BlockSpec((_BN, D_OUT_PAD), lambda i: (i, 0)),
        out_shape=jax.ShapeDtypeStruct((N, D_OUT_PAD), jnp.float32),
    )(s_prev, agg, deg)


def kernel(x, edge_index, W_self0, W_neigh0, b0,
           W_self1, W_neigh1, b1, W_self2, W_neigh2, b2):
    src = edge_index[0].astype(jnp.int32)
    dst = edge_index[1].astype(jnp.int32)
    pad = 0

    def to_flat(a, fill):
        flat = jnp.concatenate(
            [a, jnp.full((NW * E_PER_W - E,), fill, jnp.int32)])
        blocks = flat.reshape(NW, E_PER_W)
        trash = jnp.full((NW, SPAN - E_PER_W), fill, jnp.int32)
        return jnp.concatenate([blocks, trash], axis=1).reshape(NW * SPAN)

    src_p = to_flat(src, 0)
    dst_p = to_flat(dst, N)
    pad_deg = NW * CH_DEG * CHUNK - E
    dst_deg = jnp.concatenate(
        [dst, jnp.full((pad_deg,), N, jnp.int32)]).reshape(
            NW * CH_DEG, CHUNK)

    zeros64 = jnp.zeros((GCH, D_HID), jnp.float32)
    zeros128 = jnp.zeros((CHUNK, D_HID), jnp.float32)
    ones128 = jnp.ones((CHUNK, D_HID), jnp.float32)

    wn2 = jnp.pad(W_neigh2, ((0, 0), (0, D_OUT_PAD - D_OUT)))
    ws2 = jnp.pad(W_self2, ((0, 0), (0, D_OUT_PAD - D_OUT)))
    b2p = jnp.pad(b2, (0, D_OUT_PAD - D_OUT)).reshape(1, D_OUT_PAD)

    deg_parts = _make_deg()(ones128, zeros128, dst_deg)
    deg3 = deg_parts.reshape(NC, ACC_ROWS, D_HID)

    agg128 = _make_agg(D_HID)

    p0, s0 = _tc_first(x, W_neigh0, W_self0, b0.reshape(1, D_HID))
    a0 = agg128(p0, src_p, dst_p, zeros64).reshape(NC, ACC_ROWS, D_HID)

    p1, s1 = _tc_mid(s0, a0, deg3, W_neigh1, W_self1,
                     b1.reshape(1, D_HID), D_HID)
    a1 = agg128(p1, src_p, dst_p, zeros64).reshape(NC, ACC_ROWS, D_HID)

    p2, s2 = _tc_mid(s1, a1, deg3, wn2, ws2, b2p, D_OUT_PAD)
    a2 = agg128(p2, src_p, dst_p, zeros64).reshape(NC, ACC_ROWS, D_OUT_PAD)

    out = _tc_last(s2, a2, deg3)
    return out[:, :D_OUT]


# speedup vs baseline: 1.5125x; 1.3588x over previous
import functools

import jax
import jax.numpy as jnp
from jax import lax
from jax.experimental import pallas as pl
from jax.experimental.pallas import tpu as pltpu
from jax.experimental.pallas import tpu_sc as plsc

N = 10000
E = 320000
D_IN = 128
D_HID = 128
D_OUT = 47
D_OUT_PAD = 128

NC = 2
NS = 16
NW = NC * NS
CHUNK = 128

GCH = 128
NCH = 79
SPAN = NCH * GCH
E_PER_W = SPAN
E_ALLOC = NW * SPAN

ACC_ROWS = 10240
ZROWS_PER_TILE = ACC_ROWS // NS


def _make_agg(width):
    mesh = plsc.VectorSubcoreMesh(core_axis_name="c", subcore_axis_name="s")

    @functools.partial(
        pl.kernel,
        out_type=jax.ShapeDtypeStruct((NC * ACC_ROWS, width), jnp.float32),
        mesh=mesh,
        scratch_types=[
            pltpu.VMEM((GCH,), jnp.int32),
            pltpu.VMEM((GCH,), jnp.int32),
            pltpu.VMEM((GCH, width), jnp.float32),
            pltpu.VMEM_SHARED((ACC_ROWS, width), jnp.float32),
            pltpu.SemaphoreType.DMA,
        ],
    )
    def agg(p_hbm, src_hbm, dst_hbm, zeros_hbm, out_hbm,
            src_v, dst_v, rows_v, acc, sem):
        c = lax.axis_index("c")
        s = lax.axis_index("s")
        ebase = (c * NS + s) * SPAN

        pltpu.sync_copy(zeros_hbm, rows_v)
        z0 = s * ZROWS_PER_TILE
        for j in range(ZROWS_PER_TILE // GCH):
            pltpu.sync_copy(rows_v, acc.at[pl.ds(z0 + j * GCH, GCH)])
        plsc.subcore_barrier()

        def body(g, carry):
            eb = ebase + g * GCH
            pltpu.sync_copy(src_hbm.at[pl.ds(eb, GCH)], src_v)
            pltpu.sync_copy(dst_hbm.at[pl.ds(eb, GCH)], dst_v)
            pltpu.async_copy(p_hbm.at[src_v], rows_v, sem).wait()
            pltpu.sync_copy(rows_v, acc.at[dst_v], add=True)
            return carry

        lax.fori_loop(0, NCH, body, 0)
        plsc.subcore_barrier()

        for j in range(ZROWS_PER_TILE // GCH):
            pltpu.sync_copy(acc.at[pl.ds(z0 + j * GCH, GCH)], rows_v)
            pltpu.sync_copy(rows_v,
                            out_hbm.at[pl.ds(c * ACC_ROWS + z0 + j * GCH,
                                             GCH)])

    return agg


CH_DEG = 80


def _make_deg():
    mesh = plsc.VectorSubcoreMesh(core_axis_name="c", subcore_axis_name="s")

    @functools.partial(
        pl.kernel,
        out_type=jax.ShapeDtypeStruct((NC * ACC_ROWS, D_HID), jnp.float32),
        mesh=mesh,
        scratch_types=[
            pltpu.VMEM((CH_DEG, CHUNK), jnp.int32),
            pltpu.VMEM((CHUNK, D_HID), jnp.float32),
            pltpu.VMEM_SHARED((ACC_ROWS, D_HID), jnp.float32),
            pltpu.SemaphoreType.DMA,
        ],
    )
    def deg(ones_hbm, zeros_hbm, dst_hbm, out_hbm, dst_i, ones_v, acc, sem):
        c = lax.axis_index("c")
        s = lax.axis_index("s")
        wid = c * NS + s

        pltpu.sync_copy(dst_hbm.at[pl.ds(wid * CH_DEG, CH_DEG)], dst_i)
        pltpu.sync_copy(zeros_hbm, ones_v)
        z0 = s * ZROWS_PER_TILE
        drem = ZROWS_PER_TILE % CHUNK
        dzr0 = z0 + (ZROWS_PER_TILE // CHUNK) * CHUNK
        for j in range(ZROWS_PER_TILE // CHUNK):
            pltpu.sync_copy(ones_v, acc.at[pl.ds(z0 + j * CHUNK, CHUNK)])
        if drem:
            pltpu.sync_copy(ones_v.at[pl.ds(0, drem)],
                            acc.at[pl.ds(dzr0, drem)])
        pltpu.sync_copy(ones_hbm, ones_v)
        plsc.subcore_barrier()

        DB = 8

        def body(blk, carry):
            g0 = blk * DB
            for b in range(DB):
                pltpu.async_copy(ones_v, acc.at[dst_i.at[g0 + b]], sem,
                                 add=True)
            for b in range(DB):
                pltpu.make_async_copy(ones_v, acc.at[dst_i.at[g0]], sem).wait()
            return carry

        lax.fori_loop(0, CH_DEG // DB, body, 0)
        plsc.subcore_barrier()

        for j in range(ZROWS_PER_TILE // CHUNK):
            pltpu.sync_copy(acc.at[pl.ds(z0 + j * CHUNK, CHUNK)], ones_v)
            pltpu.sync_copy(ones_v,
                            out_hbm.at[pl.ds(c * ACC_ROWS + z0 + j * CHUNK,
                                             CHUNK)])
        if drem:
            pltpu.sync_copy(acc.at[pl.ds(dzr0, drem)],
                            ones_v.at[pl.ds(0, drem)])
            pltpu.sync_copy(ones_v.at[pl.ds(0, drem)],
                            out_hbm.at[pl.ds(c * ACC_ROWS + dzr0, drem)])

    return deg


_BN = 2000


def _tc_first_body(x_ref, wn_ref, ws_ref, b_ref, p_ref, s_ref):
    xb = x_ref[...]
    p_ref[...] = jnp.dot(xb, wn_ref[...], preferred_element_type=jnp.float32)
    s_ref[...] = (jnp.dot(xb, ws_ref[...], preferred_element_type=jnp.float32)
                  + b_ref[...])


def _tc_first(x, wn, ws, b):
    return pl.pallas_call(
        _tc_first_body,
        grid=(N // _BN,),
        in_specs=[
            pl.BlockSpec((_BN, D_IN), lambda i: (i, 0)),
            pl.BlockSpec((D_IN, D_HID), lambda i: (0, 0)),
            pl.BlockSpec((D_IN, D_HID), lambda i: (0, 0)),
            pl.BlockSpec((1, D_HID), lambda i: (0, 0)),
        ],
        out_specs=[
            pl.BlockSpec((_BN, D_HID), lambda i: (i, 0)),
            pl.BlockSpec((_BN, D_HID), lambda i: (i, 0)),
        ],
        out_shape=[
            jax.ShapeDtypeStruct((N, D_HID), jnp.float32),
            jax.ShapeDtypeStruct((N, D_HID), jnp.float32),
        ],
    )(x, wn, ws, b)


def _tc_mid_body(s_ref, agg_ref, deg_ref, wn_ref, ws_ref, b_ref,
                 p_ref, so_ref):
    a = agg_ref[0] + agg_ref[1]
    dcnt = deg_ref[0, :, 0:1] + deg_ref[1, :, 0:1]
    d = jnp.maximum(dcnt, 1.0)
    h = jnp.maximum(s_ref[...] + a / d, 0.0)
    p_ref[...] = jnp.dot(h, wn_ref[...], preferred_element_type=jnp.float32)
    so_ref[...] = (jnp.dot(h, ws_ref[...], preferred_element_type=jnp.float32)
                   + b_ref[...])


def _tc_mid(s_prev, agg, deg, wn, ws, b, d_out):
    return pl.pallas_call(
        _tc_mid_body,
        grid=(N // _BN,),
        in_specs=[
            pl.BlockSpec((_BN, D_HID), lambda i: (i, 0)),
            pl.BlockSpec((NC, _BN, D_HID), lambda i: (0, i, 0)),
            pl.BlockSpec((NC, _BN, D_HID), lambda i: (0, i, 0)),
            pl.BlockSpec((D_HID, d_out), lambda i: (0, 0)),
            pl.BlockSpec((D_HID, d_out), lambda i: (0, 0)),
            pl.BlockSpec((1, d_out), lambda i: (0, 0)),
        ],
        out_specs=[
            pl.BlockSpec((_BN, d_out), lambda i: (i, 0)),
            pl.BlockSpec((_BN, d_out), lambda i: (i, 0)),
        ],
        out_shape=[
            jax.ShapeDtypeStruct((N, d_out), jnp.float32),
            jax.ShapeDtypeStruct((N, d_out), jnp.float32),
        ],
    )(s_prev, agg, deg, wn, ws, b)


def _tc_last_body(s_ref, agg_ref, deg_ref, o_ref):
    a = agg_ref[0] + agg_ref[1]
    dcnt = deg_ref[0, :, 0:1] + deg_ref[1, :, 0:1]
    d = jnp.maximum(dcnt, 1.0)
    o_ref[...] = s_ref[...] + a / d


def _tc_last(s_prev, agg, deg):
    return pl.pallas_call(
        _tc_last_body,
        grid=(N // _BN,),
        in_specs=[
            pl.BlockSpec((_BN, D_OUT_PAD), lambda i: (i, 0)),
            pl.BlockSpec((NC, _BN, D_OUT_PAD), lambda i: (0, i, 0)),
            pl.BlockSpec((NC, _BN, D_HID), lambda i: (0, i, 0)),
        ],
        out_specs=pl.BlockSpec((_BN, D_OUT_PAD), lambda i: (i, 0)),
        out_shape=jax.ShapeDtypeStruct((N, D_OUT_PAD), jnp.float32),
    )(s_prev, agg, deg)


def kernel(x, edge_index, W_self0, W_neigh0, b0,
           W_self1, W_neigh1, b1, W_self2, W_neigh2, b2):
    src = edge_index[0].astype(jnp.int32)
    dst = edge_index[1].astype(jnp.int32)
    pad = 0

    def trash_rows(n):
        return N + (jnp.arange(n, dtype=jnp.int32) % (ACC_ROWS - N))

    npad = NW * E_PER_W - E
    src_p = jnp.concatenate([src, jnp.zeros((npad,), jnp.int32)])
    dst_p = jnp.concatenate([dst, trash_rows(npad)])
    pad_deg = NW * CH_DEG * CHUNK - E
    dst_deg = jnp.concatenate(
        [dst, trash_rows(pad_deg)]).reshape(NW * CH_DEG, CHUNK)

    zeros64 = jnp.zeros((GCH, D_HID), jnp.float32)
    zeros128 = jnp.zeros((CHUNK, D_HID), jnp.float32)
    ones128 = jnp.ones((CHUNK, D_HID), jnp.float32)

    wn2 = jnp.pad(W_neigh2, ((0, 0), (0, D_OUT_PAD - D_OUT)))
    ws2 = jnp.pad(W_self2, ((0, 0), (0, D_OUT_PAD - D_OUT)))
    b2p = jnp.pad(b2, (0, D_OUT_PAD - D_OUT)).reshape(1, D_OUT_PAD)

    deg_parts = _make_deg()(ones128, zeros128, dst_deg)
    deg3 = deg_parts.reshape(NC, ACC_ROWS, D_HID)

    agg128 = _make_agg(D_HID)

    p0, s0 = _tc_first(x, W_neigh0, W_self0, b0.reshape(1, D_HID))
    a0 = agg128(p0, src_p, dst_p, zeros64).reshape(NC, ACC_ROWS, D_HID)

    p1, s1 = _tc_mid(s0, a0, deg3, W_neigh1, W_self1,
                     b1.reshape(1, D_HID), D_HID)
    a1 = agg128(p1, src_p, dst_p, zeros64).reshape(NC, ACC_ROWS, D_HID)

    p2, s2 = _tc_mid(s1, a1, deg3, wn2, ws2, b2p, D_OUT_PAD)
    a2 = agg128(p2, src_p, dst_p, zeros64).reshape(NC, ACC_ROWS, D_OUT_PAD)

    out = _tc_last(s2, a2, deg3)
    return out[:, :D_OUT]
